# Initial kernel scaffold; baseline (speedup 1.0000x reference)
import jax
import jax.numpy as jnp
from jax.experimental import pallas as pl


def kernel(sims, batch_size):
    raise NotImplementedError("write your pallas kernel here")


# speedup vs baseline: 1.3686x; 1.3686x over previous
import jax
import jax.numpy as jnp
from jax.experimental import pallas as pl
import jax.experimental.pallas.tpu as pltpu


def _exp_rowsum_kernel(x_ref, s_ref, rinv_ref, acc_ref):
    j = pl.program_id(0)
    nj = pl.num_programs(0)
    s = jnp.exp(x_ref[...] * 50.0)
    s_ref[...] = s
    part = jnp.sum(s, axis=1, keepdims=True)

    @pl.when(j == 0)
    def _():
        acc_ref[...] = part

    @pl.when(j != 0)
    def _():
        acc_ref[...] += part

    @pl.when(j == nj - 1)
    def _():
        rinv_ref[...] = 1.0 / acc_ref[...]


def _col_update_kernel(s_ref, r_ref, c_ref):
    c_ref[...] = 1.0 / jnp.sum(s_ref[...] * r_ref[...], axis=0, keepdims=True)


def _row_update_kernel(s_ref, c_ref, r_ref):
    r_ref[...] = 1.0 / jnp.sum(s_ref[...] * c_ref[...], axis=1, keepdims=True)


def _final_kernel(s_ref, r_ref, out_ref, idx_ref, bv_ref, bi_ref):
    j = pl.program_id(0)
    nj = pl.num_programs(0)
    m, cb = s_ref.shape
    sr = s_ref[...] * r_ref[...]
    c = 1.0 / jnp.sum(sr, axis=0, keepdims=True)
    o = sr * c
    out_ref[...] = o
    bm = jnp.max(o, axis=1, keepdims=True)
    bi = jnp.argmax(o, axis=1).reshape(m, 1).astype(jnp.int32) + j * cb

    @pl.when(j == 0)
    def _():
        bv_ref[...] = bm
        bi_ref[...] = bi

    @pl.when(j != 0)
    def _():
        upd = bm > bv_ref[...]
        bv_ref[...] = jnp.where(upd, bm, bv_ref[...])
        bi_ref[...] = jnp.where(upd, bi, bi_ref[...])

    @pl.when(j == nj - 1)
    def _():
        idx_ref[...] = bi_ref[...]


def kernel(sims, batch_size=256):
    del batch_size
    num_row, num_col = sims.shape
    work = sims.T if num_row >= num_col else sims
    m, n = work.shape

    cb = min(512, n)
    rb = min(512, m)

    s_mat, r = pl.pallas_call(
        _exp_rowsum_kernel,
        grid=(n // cb,),
        in_specs=[pl.BlockSpec((m, cb), lambda j: (0, j))],
        out_specs=[
            pl.BlockSpec((m, cb), lambda j: (0, j)),
            pl.BlockSpec((m, 1), lambda j: (0, 0)),
        ],
        out_shape=[
            jax.ShapeDtypeStruct((m, n), jnp.float32),
            jax.ShapeDtypeStruct((m, 1), jnp.float32),
        ],
        scratch_shapes=[pltpu.VMEM((m, 1), jnp.float32)],
    )(work)

    col_update = pl.pallas_call(
        _col_update_kernel,
        grid=(n // cb,),
        in_specs=[
            pl.BlockSpec((m, cb), lambda j: (0, j)),
            pl.BlockSpec((m, 1), lambda j: (0, 0)),
        ],
        out_specs=pl.BlockSpec((1, cb), lambda j: (0, j)),
        out_shape=jax.ShapeDtypeStruct((1, n), jnp.float32),
    )

    row_update = pl.pallas_call(
        _row_update_kernel,
        grid=(m // rb,),
        in_specs=[
            pl.BlockSpec((rb, n), lambda i: (i, 0)),
            pl.BlockSpec((1, n), lambda i: (0, 0)),
        ],
        out_specs=pl.BlockSpec((rb, 1), lambda i: (i, 0)),
        out_shape=jax.ShapeDtypeStruct((m, 1), jnp.float32),
    )

    for _ in range(4):
        c = col_update(s_mat, r)
        r = row_update(s_mat, c)

    out, idx = pl.pallas_call(
        _final_kernel,
        grid=(n // cb,),
        in_specs=[
            pl.BlockSpec((m, cb), lambda j: (0, j)),
            pl.BlockSpec((m, 1), lambda j: (0, 0)),
        ],
        out_specs=[
            pl.BlockSpec((m, cb), lambda j: (0, j)),
            pl.BlockSpec((m, 1), lambda j: (0, 0)),
        ],
        out_shape=[
            jax.ShapeDtypeStruct((m, n), jnp.float32),
            jax.ShapeDtypeStruct((m, 1), jnp.int32),
        ],
        scratch_shapes=[
            pltpu.VMEM((m, 1), jnp.float32),
            pltpu.VMEM((m, 1), jnp.int32),
        ],
    )(s_mat, r)

    row_ids = jnp.arange(m, dtype=jnp.int32)
    col_ids = idx.reshape(m)
    if num_row >= num_col:
        indices = jnp.stack((col_ids, row_ids), axis=0)
    else:
        indices = jnp.stack((row_ids, col_ids), axis=0)
    values = jnp.ones((m,), dtype=jnp.float32)
    return (out, indices, values)
